# baseline (speedup 1.0000x reference)
import jax
import jax.numpy as jnp
from jax import lax
from jax.experimental import pallas as pl
from jax.experimental.pallas import tpu as pltpu

NODE = 112
CLUSTER = 8
L1 = 32
L2 = 32
L3 = 16
DN1 = 56
DN2 = 28
BLK = 8

_F32 = jnp.float32
_DEF = lax.Precision.DEFAULT
_HI = lax.Precision.HIGHEST


def _dot(a, b, dims, prec):
    return lax.dot_general(a, b, (dims, ((), ())),
                           preferred_element_type=_F32, precision=prec)


def _weights_kernel(alpha1_ref, beta1r_ref, bias1_ref, alpha2_ref, beta2_ref,
                    h2pre_ref, w2_ref, diag_scr):
    for n in range(NODE):
        diag_scr[n:n + 1, :] = _dot(alpha1_ref[n:n + 1, :],
                                    beta1r_ref[:, n, :], ((1,), (0,)), _DEF)
    ri = lax.broadcasted_iota(jnp.int32, (NODE, NODE), 0)
    ci = lax.broadcasted_iota(jnp.int32, (NODE, NODE), 1)
    eye = (ri == ci).astype(_F32)
    h2pre_ref[...] = _dot(eye, diag_scr[...], ((1,), (0,)), _DEF) + bias1_ref[...]
    w2_ref[...] = _dot(alpha2_ref[...], beta2_ref[...], ((1,), (0,)), _DEF)


def _rank_from_col(v_col, n):
    ri = lax.broadcasted_iota(jnp.int32, (n, n), 0)
    ci = lax.broadcasted_iota(jnp.int32, (n, n), 1)
    eye = (ri == ci).astype(_F32)
    bcc = jnp.broadcast_to(v_col, (n, n))
    v_row = jnp.sum(eye * bcc, axis=0, keepdims=True)
    bcr = jnp.broadcast_to(v_row, (n, n))
    beats = (bcr > v_col) | ((bcr == v_col) & (ci < ri))
    return jnp.sum(beats.astype(jnp.int32), axis=1, keepdims=True)


def _main_kernel(x_ref, h2pre_ref, w2r3_ref, bias2_ref, p1w_ref, p1b_ref,
                 p2w_ref, p2b_ref, fc1_ref, fc1b_ref, fc2_ref, fc2b_ref,
                 out_ref, hid_scr, h22_scr, x1_scr):
    h2pre = h2pre_ref[...]
    p1w = p1w_ref[...]
    p1b = p1b_ref[...]
    bias2 = bias2_ref[...]
    p2w = p2w_ref[...]
    p2b = p2b_ref[...]
    fc1 = fc1_ref[...]
    fc1b = fc1b_ref[...]
    fc2 = fc2_ref[...]
    fc2b = fc2b_ref[...]

    s1pre = _dot(x_ref[...], h2pre, ((1,), (0,)), _DEF)

    p_iota = lax.broadcasted_iota(jnp.int32, (NODE, DN1), 1)

    for s in range(BLK):
        a = x_ref[s * NODE:(s + 1) * NODE, :]
        s1 = jnp.maximum(s1pre[s * NODE:(s + 1) * NODE, :] + h2pre, 0.0)
        wcol = _dot(s1, p1w, ((1,), (0,)), _DEF) + p1b
        sig_col = jax.nn.sigmoid(wcol)
        rank_col = _rank_from_col(sig_col, NODE)
        p_selt = (rank_col == p_iota).astype(_F32)

        s1aug = jnp.concatenate([s1, sig_col], axis=1)
        hv = _dot(p_selt, s1aug, ((0,), (0,)), _HI)
        hid_scr[:, s, :] = hv[:, :L1] * hv[:, L1:L1 + 1]

        t1 = _dot(p_selt, a, ((0,), (0,)), _HI)
        t2 = _dot(t1, p_selt, ((1,), (0,)), _HI)
        adj2 = jnp.transpose(t2)
        deg = jnp.sum(adj2, axis=1, keepdims=True)
        x1_scr[s * DN1:(s + 1) * DN1, :] = adj2 / deg

    for n in range(DN1):
        h22_scr[n] = _dot(hid_scr[n], w2r3_ref[n], ((1,), (0,)), _DEF)

    for s in range(BLK):
        h22 = h22_scr[:, s, :] + bias2
        x1 = x1_scr[s * DN1:(s + 1) * DN1, :]
        s2 = jnp.maximum(_dot(x1, h22, ((1,), (0,)), _DEF) + h22, 0.0)
        wcol2 = _dot(s2, p2w, ((1,), (0,)), _DEF) + p2b
        sig2 = jax.nn.sigmoid(wcol2)
        rank2 = _rank_from_col(sig2, DN1)
        wgt = sig2 * (rank2 < DN2).astype(_F32)
        m = jnp.sum(s2 * wgt, axis=0, keepdims=True) / float(DN2)
        h3 = jnp.maximum(_dot(m, fc1, ((1,), (0,)), _DEF) + fc1b, 0.0)
        out_ref[s:s + 1, :] = _dot(h3, fc2, ((1,), (0,)), _DEF) + fc2b


@jax.jit
def kernel(x, alpha1, beta1, bias1, pool1_w, pool1_b, alpha2, beta2, bias2,
           pool2_w, pool2_b, fc1_w, fc1_b, fc2_w, fc2_b):
    b = x.shape[0]
    x2d = x.reshape(b * NODE, NODE)
    beta1r = beta1.reshape(CLUSTER, NODE, L1)

    h2pre, w2 = pl.pallas_call(
        _weights_kernel,
        in_specs=[pl.BlockSpec((NODE, CLUSTER), lambda: (0, 0)),
                  pl.BlockSpec((CLUSTER, NODE, L1), lambda: (0, 0, 0)),
                  pl.BlockSpec((1, L1), lambda: (0, 0)),
                  pl.BlockSpec((DN1, CLUSTER), lambda: (0, 0)),
                  pl.BlockSpec((CLUSTER, L1 * L2), lambda: (0, 0))],
        out_specs=[pl.BlockSpec((NODE, L1), lambda: (0, 0)),
                   pl.BlockSpec((DN1, L1 * L2), lambda: (0, 0))],
        out_shape=[jax.ShapeDtypeStruct((NODE, L1), _F32),
                   jax.ShapeDtypeStruct((DN1, L1 * L2), _F32)],
        scratch_shapes=[pltpu.VMEM((NODE, L1), _F32)],
    )(alpha1, beta1r, bias1.reshape(1, L1), alpha2, beta2)

    w2r3 = w2.reshape(DN1, L1, L2)

    out = pl.pallas_call(
        _main_kernel,
        grid=(b // BLK,),
        in_specs=[
            pl.BlockSpec((BLK * NODE, NODE), lambda i: (i, 0)),
            pl.BlockSpec((NODE, L1), lambda i: (0, 0)),
            pl.BlockSpec((DN1, L1, L2), lambda i: (0, 0, 0)),
            pl.BlockSpec((1, L2), lambda i: (0, 0)),
            pl.BlockSpec((L1, 1), lambda i: (0, 0)),
            pl.BlockSpec((1, 1), lambda i: (0, 0)),
            pl.BlockSpec((L2, 1), lambda i: (0, 0)),
            pl.BlockSpec((1, 1), lambda i: (0, 0)),
            pl.BlockSpec((L2, L3), lambda i: (0, 0)),
            pl.BlockSpec((1, L3), lambda i: (0, 0)),
            pl.BlockSpec((L3, 1), lambda i: (0, 0)),
            pl.BlockSpec((1, 1), lambda i: (0, 0)),
        ],
        out_specs=pl.BlockSpec((BLK, 1), lambda i: (i, 0)),
        out_shape=jax.ShapeDtypeStruct((b, 1), jnp.float32),
        scratch_shapes=[pltpu.VMEM((DN1, BLK, L1), _F32),
                        pltpu.VMEM((DN1, BLK, L2), _F32),
                        pltpu.VMEM((BLK * DN1, DN1), _F32)],
        compiler_params=pltpu.CompilerParams(
            dimension_semantics=("arbitrary",),
        ),
    )(x2d, h2pre, w2r3, bias2.reshape(1, L2), pool1_w, pool1_b.reshape(1, 1),
      pool2_w, pool2_b.reshape(1, 1), fc1_w, fc1_b.reshape(1, L3), fc2_w,
      fc2_b.reshape(1, 1))
    return out


# speedup vs baseline: 1.2696x; 1.2696x over previous
import jax
import jax.numpy as jnp
from jax import lax
from jax.experimental import pallas as pl
from jax.experimental.pallas import tpu as pltpu

NODE = 112
CLUSTER = 8
L1 = 32
L2 = 32
L3 = 16
DN1 = 56
DN2 = 28
BLK = 16

_F32 = jnp.float32
_DEF = lax.Precision.DEFAULT
_HI = lax.Precision.HIGHEST


def _dot(a, b, dims, prec):
    return lax.dot_general(a, b, (dims, ((), ())),
                           preferred_element_type=_F32, precision=prec)


def _weights_kernel(alpha1_ref, beta1r_ref, bias1_ref, alpha2_ref, beta2_ref,
                    h2pre_ref, w2_ref, diag_scr):
    for n in range(NODE):
        diag_scr[n:n + 1, :] = _dot(alpha1_ref[n:n + 1, :],
                                    beta1r_ref[:, n, :], ((1,), (0,)), _DEF)
    ri = lax.broadcasted_iota(jnp.int32, (NODE, NODE), 0)
    ci = lax.broadcasted_iota(jnp.int32, (NODE, NODE), 1)
    eye = (ri == ci).astype(_F32)
    h2pre_ref[...] = _dot(eye, diag_scr[...], ((1,), (0,)), _DEF) + bias1_ref[...]
    w2_ref[...] = _dot(alpha2_ref[...], beta2_ref[...], ((1,), (0,)), _DEF)


def _rank_from_col(v_col, n):
    ri = lax.broadcasted_iota(jnp.int32, (n, n), 0)
    ci = lax.broadcasted_iota(jnp.int32, (n, n), 1)
    v_row = jnp.transpose(v_col)
    bcr = jnp.broadcast_to(v_row, (n, n))
    beats = (bcr > v_col) | ((bcr == v_col) & (ci < ri))
    return jnp.sum(beats.astype(jnp.int32), axis=1, keepdims=True)


def _main_kernel(x_ref, h2pre_ref, w2r3_ref, bias2_ref, p1w_ref, p1b_ref,
                 p2w_ref, p2b_ref, fc1_ref, fc1b_ref, fc2_ref, fc2b_ref,
                 out_ref, hid_scr, h22_scr, x1_scr):
    h2pre = h2pre_ref[...]
    p1w = p1w_ref[...]
    p1b = p1b_ref[...]
    bias2 = bias2_ref[...]
    p2w = p2w_ref[...]
    p2b = p2b_ref[...]
    fc1 = fc1_ref[...]
    fc1b = fc1b_ref[...]
    fc2 = fc2_ref[...]
    fc2b = fc2b_ref[...]

    s1pre = _dot(x_ref[...], h2pre, ((1,), (0,)), _DEF)

    p_iota = lax.broadcasted_iota(jnp.int32, (NODE, DN1), 1)
    iota_row = lax.broadcasted_iota(jnp.int32, (1, NODE), 1).astype(_F32)

    for s in range(BLK):
        a = x_ref[s * NODE:(s + 1) * NODE, :]
        s1 = jnp.maximum(s1pre[s * NODE:(s + 1) * NODE, :] + h2pre, 0.0)
        wcol = _dot(s1, p1w, ((1,), (0,)), _DEF) + p1b
        sig_col = jax.nn.sigmoid(wcol)
        rank_col = _rank_from_col(sig_col, NODE)
        p_selt = (rank_col == p_iota).astype(_F32)
        idx = _dot(iota_row, p_selt, ((1,), (0,)), _DEF).astype(jnp.int32)

        s1aug_t = jnp.concatenate(
            [jnp.transpose(s1), jnp.transpose(sig_col)], axis=0)
        hv_t = jnp.take_along_axis(
            s1aug_t, jnp.broadcast_to(idx, (L1 + 1, DN1)), axis=1)
        hid_scr[:, s, :] = jnp.transpose(hv_t[:L1, :] * hv_t[L1:L1 + 1, :])

        g = jnp.take_along_axis(
            a, jnp.broadcast_to(idx, (NODE, DN1)), axis=1)
        g_t = jnp.transpose(g)
        adj2 = jnp.take_along_axis(
            g_t, jnp.broadcast_to(idx, (DN1, DN1)), axis=1)
        deg = jnp.sum(adj2, axis=1, keepdims=True)
        x1_scr[s * DN1:(s + 1) * DN1, :] = adj2 / deg

    for n in range(DN1):
        h22_scr[n] = _dot(hid_scr[n], w2r3_ref[n], ((1,), (0,)), _DEF)

    for s in range(BLK):
        h22 = h22_scr[:, s, :] + bias2
        x1 = x1_scr[s * DN1:(s + 1) * DN1, :]
        s2 = jnp.maximum(_dot(x1, h22, ((1,), (0,)), _DEF) + h22, 0.0)
        wcol2 = _dot(s2, p2w, ((1,), (0,)), _DEF) + p2b
        sig2 = jax.nn.sigmoid(wcol2)
        rank2 = _rank_from_col(sig2, DN1)
        wgt = sig2 * (rank2 < DN2).astype(_F32)
        m = jnp.sum(s2 * wgt, axis=0, keepdims=True) / float(DN2)
        h3 = jnp.maximum(_dot(m, fc1, ((1,), (0,)), _DEF) + fc1b, 0.0)
        out_ref[s:s + 1, :] = _dot(h3, fc2, ((1,), (0,)), _DEF) + fc2b


@jax.jit
def kernel(x, alpha1, beta1, bias1, pool1_w, pool1_b, alpha2, beta2, bias2,
           pool2_w, pool2_b, fc1_w, fc1_b, fc2_w, fc2_b):
    b = x.shape[0]
    x2d = x.reshape(b * NODE, NODE)
    beta1r = beta1.reshape(CLUSTER, NODE, L1)

    h2pre, w2 = pl.pallas_call(
        _weights_kernel,
        in_specs=[pl.BlockSpec((NODE, CLUSTER), lambda: (0, 0)),
                  pl.BlockSpec((CLUSTER, NODE, L1), lambda: (0, 0, 0)),
                  pl.BlockSpec((1, L1), lambda: (0, 0)),
                  pl.BlockSpec((DN1, CLUSTER), lambda: (0, 0)),
                  pl.BlockSpec((CLUSTER, L1 * L2), lambda: (0, 0))],
        out_specs=[pl.BlockSpec((NODE, L1), lambda: (0, 0)),
                   pl.BlockSpec((DN1, L1 * L2), lambda: (0, 0))],
        out_shape=[jax.ShapeDtypeStruct((NODE, L1), _F32),
                   jax.ShapeDtypeStruct((DN1, L1 * L2), _F32)],
        scratch_shapes=[pltpu.VMEM((NODE, L1), _F32)],
    )(alpha1, beta1r, bias1.reshape(1, L1), alpha2, beta2)

    w2r3 = w2.reshape(DN1, L1, L2)

    out = pl.pallas_call(
        _main_kernel,
        grid=(b // BLK,),
        in_specs=[
            pl.BlockSpec((BLK * NODE, NODE), lambda i: (i, 0)),
            pl.BlockSpec((NODE, L1), lambda i: (0, 0)),
            pl.BlockSpec((DN1, L1, L2), lambda i: (0, 0, 0)),
            pl.BlockSpec((1, L2), lambda i: (0, 0)),
            pl.BlockSpec((L1, 1), lambda i: (0, 0)),
            pl.BlockSpec((1, 1), lambda i: (0, 0)),
            pl.BlockSpec((L2, 1), lambda i: (0, 0)),
            pl.BlockSpec((1, 1), lambda i: (0, 0)),
            pl.BlockSpec((L2, L3), lambda i: (0, 0)),
            pl.BlockSpec((1, L3), lambda i: (0, 0)),
            pl.BlockSpec((L3, 1), lambda i: (0, 0)),
            pl.BlockSpec((1, 1), lambda i: (0, 0)),
        ],
        out_specs=pl.BlockSpec((BLK, 1), lambda i: (i, 0)),
        out_shape=jax.ShapeDtypeStruct((b, 1), jnp.float32),
        scratch_shapes=[pltpu.VMEM((DN1, BLK, L1), _F32),
                        pltpu.VMEM((DN1, BLK, L2), _F32),
                        pltpu.VMEM((BLK * DN1, DN1), _F32)],
        compiler_params=pltpu.CompilerParams(
            dimension_semantics=("arbitrary",),
        ),
    )(x2d, h2pre, w2r3, bias2.reshape(1, L2), pool1_w, pool1_b.reshape(1, 1),
      pool2_w, pool2_b.reshape(1, 1), fc1_w, fc1_b.reshape(1, L3), fc2_w,
      fc2_b.reshape(1, 1))
    return out
